# baseline (speedup 1.0000x reference)
import re

import jax
import jax.numpy as jnp
from jax.experimental import pallas as pl


def _probe():
    import reference as r
    try:
        dev = jax.devices()[0]
        if dev.platform != "tpu":
            return
    except Exception:
        return
    d = r.setup_inputs(0)
    jax.block_until_ready(d)
    for k, v in d.items():
        try:
            fmt = v.format
        except Exception as e:
            fmt = f"err {e}"
        print(f"[probe] {k}: shape={v.shape} dtype={v.dtype} fmt={fmt}")
    inputs = (d["dense"], d["sparse"], d["tables"])
    comp = jax.jit(r.reference).lower(*inputs).compile()
    txt = comp.as_text()
    m = re.search(r"ENTRY[^\n]*\n((?:[^\n]*parameter[^\n]*\n)*)", txt)
    for line in txt.splitlines():
        if "parameter(" in line and "=" in line:
            print("[probe-ref-entry]", line.strip()[:160])
        if line.startswith("ENTRY"):
            print("[probe-ref-entry]", line.strip()[:200])
    del m


_probe()


@jax.jit
def kernel(dense, sparse, tables):
    f = jnp.arange(26)[None, :]
    gathered = tables[f, sparse]
    return jnp.concatenate([dense, gathered.reshape(16384, 26 * 32)], axis=-1)


# speedup vs baseline: 1.5518x; 1.5518x over previous
import functools

import jax
import jax.numpy as jnp
from jax import lax
from jax.experimental import pallas as pl
from jax.experimental.pallas import tpu as pltpu
from jax.experimental.pallas import tpu_sc as plsc

B = 16384
F = 26
V = 100000
E = 32
DD = 64
OUT = DD + F * E
FE = F * E

NC, NS, L = 2, 16, 16
NW = NC * NS
PAIRS_W = FE // NW
BH = B // 2


def _sc_body(sparse_hbm, tables_hbm, inter_hbm, erow_v, sp_v, col_v, semw):
    cid = lax.axis_index("c")
    sid = lax.axis_index("s")
    wid = sid * NC + cid

    VA = (V // 128) * 128

    def pair_body(j, carry):
        fe = wid * PAIRS_W + j
        f = fe // E
        pltpu.sync_copy(tables_hbm.at[fe], erow_v)

        def half_body(h, carry2):
            b0 = h * BH
            pltpu.sync_copy(sparse_hbm.at[f, pl.ds(b0, BH)], sp_v)

            def group(o, carry3):
                idx = sp_v[pl.ds(o * L, L)]
                col_v[pl.ds(o * L, L)] = plsc.load_gather(erow_v, [idx])
                return carry3

            lax.fori_loop(0, BH // L, group, 0)
            pltpu.sync_copy(col_v, inter_hbm.at[fe, pl.ds(b0, BH)])
            return carry2

        lax.fori_loop(0, 2, half_body, 0)
        return carry

    lax.fori_loop(0, PAIRS_W, pair_body, 0)


_sc_gather = functools.partial(
    pl.kernel,
    out_type=jax.ShapeDtypeStruct((FE, B), jnp.float32),
    mesh=plsc.VectorSubcoreMesh(core_axis_name="c", subcore_axis_name="s",
                                num_cores=NC, num_subcores=NS),
    compiler_params=pltpu.CompilerParams(needs_layout_passes=False),
    scratch_types=[
        pltpu.VMEM((V,), jnp.float32),
        pltpu.VMEM((BH,), jnp.int32),
        pltpu.VMEM((BH,), jnp.float32),
        pltpu.SemaphoreType.DMA,
    ],
)(_sc_body)


ROWS_TC = 128


def _tc_body(dense_ref, inter_ref, out_ref):
    out_ref[:, 0:DD] = dense_ref[...]
    out_ref[:, DD:OUT] = inter_ref[...].T


_tc_combine = functools.partial(
    pl.pallas_call,
    grid=(B // ROWS_TC,),
    in_specs=[
        pl.BlockSpec((ROWS_TC, DD), lambda i: (i, 0)),
        pl.BlockSpec((FE, ROWS_TC), lambda i: (0, i)),
    ],
    out_specs=pl.BlockSpec((ROWS_TC, OUT), lambda i: (i, 0)),
    out_shape=jax.ShapeDtypeStruct((B, OUT), jnp.float32),
)(_tc_body)


@jax.jit
def kernel(dense, sparse, tables):
    tables_t = tables.transpose(0, 2, 1).reshape(FE, V)
    sparse_t = sparse.astype(jnp.int32).T
    inter = _sc_gather(sparse_t, tables_t)
    return _tc_combine(dense, inter)
